# Initial kernel scaffold; baseline (speedup 1.0000x reference)
import functools

import jax
import jax.numpy as jnp
from jax import lax
from jax.experimental import pallas as pl
from jax.experimental.pallas import tpu as pltpu
from jax.experimental.pallas import tpu_sc as plsc

N = 10000
E = 160000
R = 50
D = 128
B = 8

NC, NS, L = 2, 16, 16
NW = NC * NS
NPAD = 10240
EPAD = 163840
CH = 128
ROWS_W = EPAD // NW // CH
NODES_T = NPAD // NS

BN = 1000


def _hr_body(a_ref, V_ref, h_ref, out_ref, W_s):
    r = pl.program_id(0)

    @pl.when(pl.program_id(1) == 0)
    def _():
        W = a_ref[r, 0] * V_ref[0]
        for b in range(1, B):
            W = W + a_ref[r, b] * V_ref[b]
        W_s[...] = W

    out_ref[...] = jnp.dot(h_ref[...], W_s[...],
                           preferred_element_type=jnp.float32)


def _hr_table(h, a, V):
    return pl.pallas_call(
        _hr_body,
        grid=(R, N // BN),
        in_specs=[
            pl.BlockSpec(memory_space=pltpu.SMEM),
            pl.BlockSpec((B, D, D), lambda r, n: (0, 0, 0)),
            pl.BlockSpec((BN, D), lambda r, n: (n, 0)),
        ],
        out_specs=pl.BlockSpec((BN, D), lambda r, n: (n, r)),
        out_shape=jax.ShapeDtypeStruct((N, R * D), jnp.float32),
        scratch_shapes=[pltpu.VMEM((D, D), jnp.float32)],
    )(a, V, h)


def _make_sc_edges(with_deg):
    mesh = plsc.VectorSubcoreMesh(core_axis_name="c", subcore_axis_name="s")
    out_type = [jax.ShapeDtypeStruct((NC, NPAD, D), jnp.float32)]
    if with_deg:
        out_type.append(jax.ShapeDtypeStruct((NC, NPAD, L), jnp.float32))
    scratch_types = (
        pltpu.VMEM((ROWS_W, CH), jnp.int32),
        pltpu.VMEM((ROWS_W, CH), jnp.int32),
        pltpu.VMEM((ROWS_W, CH), jnp.int32),
        pltpu.VMEM((ROWS_W, CH), jnp.int32),
        pltpu.VMEM((CH, D), jnp.float32),
        pltpu.VMEM((CH, L), jnp.float32),
        pltpu.VMEM_SHARED((NPAD, D), jnp.float32),
        pltpu.VMEM_SHARED((NPAD, L), jnp.float32),
        pltpu.SemaphoreType.DMA,
    )

    def body(hr, srcr, dstr, etr, z128r, z16r, ones16r, *rest):
        if with_deg:
            agg_out, deg_out = rest[0], rest[1]
            rest = rest[2:]
        else:
            agg_out = rest[0]
            deg_out = None
            rest = rest[1:]
        (src_s, dst_s, et_s, flat_s, rows_v, ones_v,
         agg_sh, deg_sh, sem) = rest
        c = lax.axis_index("c")
        s = lax.axis_index("s")
        w = c * NS + s
        tb = s * NODES_T

        pltpu.sync_copy(z128r, agg_sh.at[pl.ds(tb, NODES_T)])
        if with_deg:
            pltpu.sync_copy(z16r, deg_sh.at[pl.ds(tb, NODES_T)])
            pltpu.sync_copy(ones16r, ones_v)

        erow = w * ROWS_W
        pltpu.sync_copy(srcr.at[pl.ds(erow, ROWS_W)], src_s)
        pltpu.sync_copy(dstr.at[pl.ds(erow, ROWS_W)], dst_s)
        pltpu.sync_copy(etr.at[pl.ds(erow, ROWS_W)], et_s)

        def fbody(j, carry):
            for k in range(CH // L):
                sl = pl.ds(k * L, L)
                flat_s[j, sl] = src_s[j, sl] * R + et_s[j, sl]
            return carry

        lax.fori_loop(0, ROWS_W, fbody, 0)
        plsc.subcore_barrier()

        def cbody(j, carry):
            pltpu.async_copy(hr.at[flat_s.at[j]], rows_v, sem).wait()
            pltpu.sync_copy(rows_v, agg_sh.at[dst_s.at[j]], add=True)
            if with_deg:
                pltpu.sync_copy(ones_v, deg_sh.at[dst_s.at[j]], add=True)
            return carry

        lax.fori_loop(0, ROWS_W, cbody, 0)
        plsc.subcore_barrier()

        pltpu.sync_copy(agg_sh.at[pl.ds(tb, NODES_T)],
                        agg_out.at[c, pl.ds(tb, NODES_T)])
        if with_deg:
            pltpu.sync_copy(deg_sh.at[pl.ds(tb, NODES_T)],
                            deg_out.at[c, pl.ds(tb, NODES_T)])

    def entry(hr, srcr, dstr, etr, z128r, z16r, ones16r):
        return pl.kernel(
            body,
            out_type=tuple(out_type),
            mesh=mesh,
            scratch_types=scratch_types,
        )(hr, srcr, dstr, etr, z128r, z16r, ones16r)

    return entry


_sc_edges_deg = _make_sc_edges(True)
_sc_edges = _make_sc_edges(False)


def _comb_body(agg_ref, deg_ref, h_ref, loop_ref, b_ref, out_ref, *, relu):
    agg = agg_ref[0] + agg_ref[1]
    deg = deg_ref[0][:, 0:1] + deg_ref[1][:, 0:1]
    norm = 1.0 / jnp.maximum(deg, 1.0)
    res = agg * norm + jnp.dot(h_ref[...], loop_ref[...],
                               preferred_element_type=jnp.float32) + b_ref[...]
    out_ref[...] = jnp.maximum(res, 0.0) if relu else res


def _comb(aggP, degw, h, loop, b2d, relu):
    return pl.pallas_call(
        functools.partial(_comb_body, relu=relu),
        grid=(N // BN,),
        in_specs=[
            pl.BlockSpec((NC, BN, D), lambda n: (0, n, 0)),
            pl.BlockSpec((NC, BN, L), lambda n: (0, n, 0)),
            pl.BlockSpec((BN, D), lambda n: (n, 0)),
            pl.BlockSpec((D, D), lambda n: (0, 0)),
            pl.BlockSpec((1, D), lambda n: (0, 0)),
        ],
        out_specs=pl.BlockSpec((BN, D), lambda n: (n, 0)),
        out_shape=jax.ShapeDtypeStruct((N, D), jnp.float32),
    )(aggP, degw, h, loop, b2d)


def kernel(x, edge_index, edge_type, V1, a1, loop1, b1, V2, a2, loop2, b2):
    src = edge_index[0].astype(jnp.int32)
    dst = edge_index[1].astype(jnp.int32)
    et = edge_type.astype(jnp.int32)
    pad = EPAD - E
    src_p = jnp.pad(src, (0, pad)).reshape(EPAD // CH, CH)
    dst_p = jnp.pad(dst, (0, pad), constant_values=N).reshape(EPAD // CH, CH)
    et_p = jnp.pad(et, (0, pad)).reshape(EPAD // CH, CH)
    z128 = jnp.zeros((NODES_T, D), jnp.float32)
    z16 = jnp.zeros((NODES_T, L), jnp.float32)
    ones16 = jnp.ones((CH, L), jnp.float32)

    hr1 = _hr_table(x, a1, V1).reshape(N * R, D)
    aggP1, degw = _sc_edges_deg(hr1, src_p, dst_p, et_p, z128, z16, ones16)
    h1 = _comb(aggP1, degw, x, loop1, b1.reshape(1, D), relu=True)
    hr2 = _hr_table(h1, a2, V2).reshape(N * R, D)
    (aggP2,) = _sc_edges(hr2, src_p, dst_p, et_p, z128, z16, ones16)
    out = _comb(aggP2, degw, h1, loop2, b2.reshape(1, D), relu=False)
    return out


# speedup vs baseline: 2.3628x; 2.3628x over previous
import functools

import jax
import jax.numpy as jnp
from jax import lax
from jax.experimental import pallas as pl
from jax.experimental.pallas import tpu as pltpu
from jax.experimental.pallas import tpu_sc as plsc

N = 10000
E = 160000
R = 50
D = 128
B = 8

NC, NS, L = 2, 16, 16
NW = NC * NS
NPAD = 10240
EPAD = 163840
CH = 128
ROWS_W = EPAD // NW // CH
NODES_T = NPAD // NS

BN = 1000


def _hr_body(a_ref, V_ref, h_ref, out_ref, W_s):
    r = pl.program_id(0)

    @pl.when(pl.program_id(1) == 0)
    def _():
        W = a_ref[r, 0] * V_ref[0]
        for b in range(1, B):
            W = W + a_ref[r, b] * V_ref[b]
        W_s[...] = W

    out_ref[...] = jnp.dot(h_ref[...], W_s[...],
                           preferred_element_type=jnp.float32)


def _hr_table(h, a, V):
    return pl.pallas_call(
        _hr_body,
        grid=(R, N // BN),
        in_specs=[
            pl.BlockSpec(memory_space=pltpu.SMEM),
            pl.BlockSpec((B, D, D), lambda r, n: (0, 0, 0)),
            pl.BlockSpec((BN, D), lambda r, n: (n, 0)),
        ],
        out_specs=pl.BlockSpec((BN, D), lambda r, n: (n, r)),
        out_shape=jax.ShapeDtypeStruct((N, R * D), jnp.float32),
        scratch_shapes=[pltpu.VMEM((D, D), jnp.float32)],
    )(a, V, h)


def _sc_mesh():
    return plsc.VectorSubcoreMesh(core_axis_name="c", subcore_axis_name="s",
                                  num_cores=NC, num_subcores=NS)


def _edges_body(hr, srcr, dstr, etr, z128r, agg_out,
                dst_s, et_s, flat_s, rows_v, agg_sh, sem):
    c = lax.axis_index("c")
    s = lax.axis_index("s")
    w = c * NS + s
    tb = s * NODES_T

    pltpu.sync_copy(z128r, agg_sh.at[pl.ds(tb, NODES_T)])

    erow = w * ROWS_W
    pltpu.sync_copy(srcr.at[pl.ds(erow, ROWS_W)], flat_s)
    pltpu.sync_copy(dstr.at[pl.ds(erow, ROWS_W)], dst_s)
    pltpu.sync_copy(etr.at[pl.ds(erow, ROWS_W)], et_s)

    def fbody(j, carry):
        for k in range(CH // L):
            sl = pl.ds(k * L, L)
            flat_s[j, sl] = flat_s[j, sl] * R + et_s[j, sl]
        return carry

    lax.fori_loop(0, ROWS_W, fbody, 0)
    plsc.subcore_barrier()

    def cbody(j, carry):
        pltpu.async_copy(hr.at[flat_s.at[j]], rows_v, sem).wait()
        pltpu.sync_copy(rows_v, agg_sh.at[dst_s.at[j]], add=True)
        return carry

    lax.fori_loop(0, ROWS_W, cbody, 0)
    plsc.subcore_barrier()

    pltpu.sync_copy(agg_sh.at[pl.ds(tb, NODES_T)],
                    agg_out.at[c, pl.ds(tb, NODES_T)])


def _sc_edges(hr, srcr, dstr, etr, z128r):
    return pl.kernel(
        _edges_body,
        out_type=jax.ShapeDtypeStruct((NC, NPAD, D), jnp.float32),
        mesh=_sc_mesh(),
        scratch_types=(
            pltpu.VMEM((ROWS_W, CH), jnp.int32),
            pltpu.VMEM((ROWS_W, CH), jnp.int32),
            pltpu.VMEM((ROWS_W, CH), jnp.int32),
            pltpu.VMEM((CH, D), jnp.float32),
            pltpu.VMEM_SHARED((NPAD, D), jnp.float32),
            pltpu.SemaphoreType.DMA,
        ),
    )(hr, srcr, dstr, etr, z128r)


def _deg_body(dstr, z128r, ones_r, deg_out, dst_s, ones_v, deg_sh):
    c = lax.axis_index("c")
    s = lax.axis_index("s")
    w = c * NS + s
    tb = s * NODES_T

    pltpu.sync_copy(z128r, deg_sh.at[pl.ds(tb, NODES_T)])
    pltpu.sync_copy(ones_r, ones_v)
    pltpu.sync_copy(dstr.at[pl.ds(w * ROWS_W, ROWS_W)], dst_s)
    plsc.subcore_barrier()

    def cbody(j, carry):
        pltpu.sync_copy(ones_v, deg_sh.at[dst_s.at[j]], add=True)
        return carry

    lax.fori_loop(0, ROWS_W, cbody, 0)
    plsc.subcore_barrier()
    pltpu.sync_copy(deg_sh.at[pl.ds(tb, NODES_T)],
                    deg_out.at[c, pl.ds(tb, NODES_T)])


def _sc_deg(dstr, z128r, ones_r):
    return pl.kernel(
        _deg_body,
        out_type=jax.ShapeDtypeStruct((NC, NPAD, D), jnp.float32),
        mesh=_sc_mesh(),
        scratch_types=(
            pltpu.VMEM((ROWS_W, CH), jnp.int32),
            pltpu.VMEM((CH, D), jnp.float32),
            pltpu.VMEM_SHARED((NPAD, D), jnp.float32),
        ),
    )(dstr, z128r, ones_r)


def _comb_body(agg_ref, deg_ref, h_ref, loop_ref, b_ref, out_ref, *, relu):
    agg = agg_ref[0] + agg_ref[1]
    deg = deg_ref[0][:, 0:1] + deg_ref[1][:, 0:1]
    norm = 1.0 / jnp.maximum(deg, 1.0)
    res = agg * norm + jnp.dot(h_ref[...], loop_ref[...],
                               preferred_element_type=jnp.float32) + b_ref[...]
    out_ref[...] = jnp.maximum(res, 0.0) if relu else res


def _comb(aggP, degw, h, loop, b2d, relu):
    return pl.pallas_call(
        functools.partial(_comb_body, relu=relu),
        grid=(N // BN,),
        in_specs=[
            pl.BlockSpec((NC, BN, D), lambda n: (0, n, 0)),
            pl.BlockSpec((NC, BN, D), lambda n: (0, n, 0)),
            pl.BlockSpec((BN, D), lambda n: (n, 0)),
            pl.BlockSpec((D, D), lambda n: (0, 0)),
            pl.BlockSpec((1, D), lambda n: (0, 0)),
        ],
        out_specs=pl.BlockSpec((BN, D), lambda n: (n, 0)),
        out_shape=jax.ShapeDtypeStruct((N, D), jnp.float32),
    )(aggP, degw, h, loop, b2d)


def kernel(x, edge_index, edge_type, V1, a1, loop1, b1, V2, a2, loop2, b2):
    src = edge_index[0].astype(jnp.int32)
    dst = edge_index[1].astype(jnp.int32)
    et = edge_type.astype(jnp.int32)
    pad = EPAD - E
    src_p = jnp.pad(src, (0, pad)).reshape(EPAD // CH, CH)
    dst_p = jnp.pad(dst, (0, pad), constant_values=N).reshape(EPAD // CH, CH)
    et_p = jnp.pad(et, (0, pad)).reshape(EPAD // CH, CH)
    z128 = jnp.zeros((NODES_T, D), jnp.float32)
    ones128 = jnp.ones((CH, D), jnp.float32)

    degw = _sc_deg(dst_p, z128, ones128)
    hr1 = _hr_table(x, a1, V1).reshape(N * R, D)
    aggP1 = _sc_edges(hr1, src_p, dst_p, et_p, z128)
    h1 = _comb(aggP1, degw, x, loop1, b1.reshape(1, D), relu=True)
    hr2 = _hr_table(h1, a2, V2).reshape(N * R, D)
    aggP2 = _sc_edges(hr2, src_p, dst_p, et_p, z128)
    out = _comb(aggP2, degw, h1, loop2, b2.reshape(1, D), relu=False)
    return out
